# Initial kernel scaffold; baseline (speedup 1.0000x reference)
import jax
import jax.numpy as jnp
from jax import lax
from jax.experimental import pallas as pl
from jax.experimental.pallas import tpu as pltpu
from jax.experimental.pallas import tpu_sc as plsc

NUM_PRIM = 8192
DIM = 64
BATCH = 64
HW = 1024
N = BATCH * HW
NC, NS = 2, 16
NW = NC * NS
CHUNK = 128
PER_W = N // NW
NCH = PER_W // CHUNK
HPER = N // NS
HCH = HPER // CHUNK
CSLICE = NUM_PRIM // NS
LANES = 16


def _sc_body(idx_g, idx_h, table, act, emb_out, cnt_out,
             idx_v, hidx_v, rows_v, ones_v, acc_v, act_v, hist_sh,
             sem0, sem1):
    c = lax.axis_index("c")
    s = lax.axis_index("s")
    wid = s * NC + c
    on_c0 = c == 0

    pltpu.sync_copy(idx_g.at[wid], idx_v)

    @pl.when(on_c0)
    def _stage_hist():
        pltpu.sync_copy(idx_h.at[s], hidx_v)
        one = jnp.ones((LANES,), jnp.int32)
        zero = jnp.zeros((LANES,), jnp.int32)
        for i in range(CHUNK // LANES):
            ones_v[pl.ds(i * LANES, LANES)] = one
        for i in range(CSLICE // LANES):
            acc_v[pl.ds(i * LANES, LANES)] = zero
        pltpu.sync_copy(acc_v, hist_sh.at[pl.ds(s * CSLICE, CSLICE)])

    plsc.subcore_barrier()

    @pl.when(on_c0)
    def _histogram():
        def hstep(j, carry):
            pltpu.sync_copy(ones_v, hist_sh.at[hidx_v.at[j]], add=True)
            return carry
        lax.fori_loop(0, HCH, hstep, 0)

    my_out = emb_out.at[wid]

    def gpair(p, carry):
        k0 = p * 2
        d0 = pltpu.async_copy(table.at[idx_v.at[k0]], rows_v.at[0], sem0)
        d1 = pltpu.async_copy(table.at[idx_v.at[k0 + 1]], rows_v.at[1], sem1)
        d0.wait()
        pltpu.sync_copy(rows_v.at[0], my_out.at[k0])
        d1.wait()
        pltpu.sync_copy(rows_v.at[1], my_out.at[k0 + 1])
        return carry

    lax.fori_loop(0, NCH // 2, gpair, 0)

    plsc.subcore_barrier()

    @pl.when(on_c0)
    def _finalize_counts():
        sl = pl.ds(s * CSLICE, CSLICE)
        pltpu.sync_copy(hist_sh.at[sl], acc_v)
        pltpu.sync_copy(act.at[sl], act_v)
        for i in range(CSLICE // LANES):
            vsl = pl.ds(i * LANES, LANES)
            acc_v[vsl] = acc_v[vsl] + act_v[vsl]
        pltpu.sync_copy(acc_v, cnt_out.at[sl])


_sc_kernel = pl.kernel(
    _sc_body,
    out_type=(
        jax.ShapeDtypeStruct((NW, NCH, CHUNK, DIM), jnp.float32),
        jax.ShapeDtypeStruct((NUM_PRIM,), jnp.int32),
    ),
    mesh=plsc.VectorSubcoreMesh(
        core_axis_name="c", subcore_axis_name="s",
        num_cores=NC, num_subcores=NS,
    ),
    scratch_types=[
        pltpu.VMEM((NCH, CHUNK), jnp.int32),
        pltpu.VMEM((HCH, CHUNK), jnp.int32),
        pltpu.VMEM((2, CHUNK, DIM), jnp.float32),
        pltpu.VMEM((CHUNK,), jnp.int32),
        pltpu.VMEM((CSLICE,), jnp.int32),
        pltpu.VMEM((CSLICE,), jnp.int32),
        pltpu.VMEM_SHARED((NUM_PRIM,), jnp.int32),
        pltpu.SemaphoreType.DMA,
        pltpu.SemaphoreType.DMA,
    ],
)


@jax.jit
def kernel(indices, primitives, activation_count):
    flat = indices.reshape(-1)
    idx_g = flat.reshape(NW, NCH, CHUNK)
    idx_h = flat.reshape(NS, HCH, CHUNK)
    emb, cnt = _sc_kernel(idx_g, idx_h, primitives, activation_count)
    return emb.reshape(BATCH, HW, DIM), cnt


# speedup vs baseline: 2.9987x; 2.9987x over previous
import jax
import jax.numpy as jnp
from jax import lax
from jax.experimental import pallas as pl
from jax.experimental.pallas import tpu as pltpu
from jax.experimental.pallas import tpu_sc as plsc

NUM_PRIM = 8192
DIM = 64
BATCH = 64
HW = 1024
N = BATCH * HW
NC, NS = 2, 16
NW = NC * NS
CHUNK = 128
PER_W = N // NW
NCH = PER_W // CHUNK
HPER = N // NS
HCH = HPER // CHUNK
CSLICE = NUM_PRIM // NS
LANES = 16


def _sc_body(idx_g, table, act, emb_out, cnt_out,
             idx_v, hidx_v, rows_v, ones_v, acc_v, act_v, hist_sh,
             sem0, sem1):
    c = lax.axis_index("c")
    s = lax.axis_index("s")
    wid = s * NC + c
    on_c0 = c == 0

    pltpu.sync_copy(idx_g.at[wid], idx_v)

    @pl.when(on_c0)
    def _stage_hist():
        pltpu.sync_copy(idx_g.at[2 * s], hidx_v.at[pl.ds(0, NCH)])
        pltpu.sync_copy(idx_g.at[2 * s + 1], hidx_v.at[pl.ds(NCH, NCH)])
        one = jnp.ones((LANES,), jnp.int32)
        zero = jnp.zeros((LANES,), jnp.int32)
        for i in range(CHUNK // LANES):
            ones_v[pl.ds(i * LANES, LANES)] = one
        for i in range(CSLICE // LANES):
            acc_v[pl.ds(i * LANES, LANES)] = zero
        pltpu.sync_copy(acc_v, hist_sh.at[pl.ds(s * CSLICE, CSLICE)])

    plsc.subcore_barrier()

    @pl.when(on_c0)
    def _histogram():
        def hstep(j, carry):
            pltpu.sync_copy(ones_v, hist_sh.at[hidx_v.at[j]], add=True)
            return carry
        lax.fori_loop(0, HCH, hstep, 0)

    my_out = emb_out.at[wid]

    def gpair(p, carry):
        k0 = p * 2
        d0 = pltpu.async_copy(table.at[idx_v.at[k0]], rows_v.at[0], sem0)
        d1 = pltpu.async_copy(table.at[idx_v.at[k0 + 1]], rows_v.at[1], sem1)
        d0.wait()
        pltpu.sync_copy(rows_v.at[0], my_out.at[k0])
        d1.wait()
        pltpu.sync_copy(rows_v.at[1], my_out.at[k0 + 1])
        return carry

    lax.fori_loop(0, NCH // 2, gpair, 0)

    plsc.subcore_barrier()

    @pl.when(on_c0)
    def _finalize_counts():
        sl = pl.ds(s * CSLICE, CSLICE)
        pltpu.sync_copy(hist_sh.at[sl], acc_v)
        pltpu.sync_copy(act.at[sl], act_v)
        for i in range(CSLICE // LANES):
            vsl = pl.ds(i * LANES, LANES)
            acc_v[vsl] = acc_v[vsl] + act_v[vsl]
        pltpu.sync_copy(acc_v, cnt_out.at[sl])


_sc_kernel = pl.kernel(
    _sc_body,
    out_type=(
        jax.ShapeDtypeStruct((NW, NCH, CHUNK, DIM), jnp.float32),
        jax.ShapeDtypeStruct((NUM_PRIM,), jnp.int32),
    ),
    mesh=plsc.VectorSubcoreMesh(
        core_axis_name="c", subcore_axis_name="s",
        num_cores=NC, num_subcores=NS,
    ),
    compiler_params=pltpu.CompilerParams(use_tc_tiling_on_sc=False),
    scratch_types=[
        pltpu.VMEM((NCH, CHUNK), jnp.int32),
        pltpu.VMEM((HCH, CHUNK), jnp.int32),
        pltpu.VMEM((2, CHUNK, DIM), jnp.float32),
        pltpu.VMEM((CHUNK,), jnp.int32),
        pltpu.VMEM((CSLICE,), jnp.int32),
        pltpu.VMEM((CSLICE,), jnp.int32),
        pltpu.VMEM_SHARED((NUM_PRIM,), jnp.int32),
        pltpu.SemaphoreType.DMA,
        pltpu.SemaphoreType.DMA,
    ],
)


@jax.jit
def kernel(indices, primitives, activation_count):
    idx_g = indices.reshape(NW, NCH, CHUNK)
    emb, cnt = _sc_kernel(idx_g, primitives, activation_count)
    return emb.reshape(BATCH, HW, DIM), cnt
